# baseline (speedup 1.0000x reference)
import jax
import jax.numpy as jnp
from jax import lax
from jax.experimental import pallas as pl
from jax.experimental.pallas import tpu as pltpu
from jax.experimental.pallas import tpu_sc as plsc

C = 64
H = 496
W = 432
HP = 512
P = 40000
B = 4
N = B * W * HP
NC = 2
NS = 16
NW = NC * NS
RANGE = N // NW
KCH = 8000
NKCH = P // KCH
CW = 128
XPB = 24
T = XPB * HP
GPB = W // XPB
L = 16
DUMP = N

_SC_PARAMS = pltpu.CompilerParams(needs_layout_passes=False)


def _k1_body(key_hbm, ptr_hbm, keys_v, lmap):
  wid = lax.axis_index("s") * NC + lax.axis_index("c")
  kbase = wid * RANGE
  iota = lax.iota(jnp.int32, L)

  def zero_body(i, _):
    lmap[pl.ds(i * L, L)] = jnp.zeros((L,), jnp.int32)
    return 0
  lax.fori_loop(0, RANGE // L, zero_body, 0)

  GV = 10
  for ci in range(NKCH):
    pltpu.sync_copy(key_hbm.at[pl.ds(ci * KCH, KCH)], keys_v)

    def group_body(gi, _, ci=ci):
      def vreg(u, gi=gi):
        i = gi * GV + u
        k16 = keys_v[pl.ds(i * L, L)]
        pv = ci * KCH + i * L + iota + 1
        inr = (k16 >= kbase) & (k16 < kbase + RANGE)
        kk = jnp.where(inr, k16 - kbase, 0)
        return k16, pv, inr, kk

      acc = jnp.zeros((L,), jnp.bool_)
      for u in range(GV):
        _, pv, inr, kk = vreg(u)
        plsc.store_scatter(lmap, [kk], pv, mask=inr)
        g = plsc.load_gather(lmap, [kk])
        acc = acc | (inr & (g < pv))

      def fix(_):
        for u in range(GV):
          _, pv, inr, kk = vreg(u)

          def cond(lost_c):
            return jnp.any(lost_c)

          def step(lost_c, pv=pv, inr=inr, kk=kk):
            plsc.store_scatter(lmap, [kk], pv, mask=lost_c)
            g2 = plsc.load_gather(lmap, [kk])
            return inr & (g2 < pv)
          g = plsc.load_gather(lmap, [kk])
          lax.while_loop(cond, step, inr & (g < pv))
        return 0
      lax.cond(plsc.all_reduce_population_count(acc)[0] > 0,
               fix, lambda _: 0, 0)
      return 0
    lax.fori_loop(0, KCH // L // GV, group_body, 0)

  pltpu.sync_copy(lmap, ptr_hbm.at[pl.ds(kbase, RANGE)])


def _k2_body(feat_hbm, key_hbm, ptr_hbm, canvas_hbm,
             keys2, w2, ridx2, featbuf, sem_k, sem_f, sem_g, sem_s):
  wid = lax.axis_index("s") * NC + lax.axis_index("c")
  iota = lax.iota(jnp.int32, L)

  def superchunk(base, rows, tail64):
    kc = []
    for j in range(rows):
      cs = 64 if (tail64 and j == rows - 1) else 128
      kc.append(pltpu.async_copy(key_hbm.at[pl.ds(base + j * 128, cs)],
                                 keys2.at[j, pl.ds(0, cs)], sem_k))
    if tail64:
      for v in range(4):
        keys2[rows - 1, pl.ds(64 + v * L, L)] = jnp.zeros((L,), jnp.int32)
    npil = rows * 128 - (64 if tail64 else 0)
    cf = pltpu.async_copy(feat_hbm.at[pl.ds(base, npil), :],
                          featbuf.at[pl.ds(0, npil), :], sem_f)
    for c in kc:
      c.wait()
    cg = [pltpu.async_copy(ptr_hbm.at[keys2.at[j]], w2.at[j], sem_g)
          for j in range(rows)]
    for c in cg:
      c.wait()
    for j in range(rows):
      for v in range(128 // L):
        k16 = keys2[j, pl.ds(v * L, L)]
        w16 = w2[j, pl.ds(v * L, L)]
        p16 = base + j * 128 + v * L + iota + 1
        ridx2[j, pl.ds(v * L, L)] = jnp.where(w16 == p16, k16, DUMP)
    cf.wait()
    cs = [pltpu.async_copy(featbuf.at[pl.ds(j * 128, 128), :],
                           canvas_hbm.at[ridx2.at[j]], sem_s)
          for j in range(rows)]
    for c in cs:
      c.wait()

  @pl.when(wid < NW - 1)
  def _():
    superchunk(wid * 1280, 5, False)
    superchunk(wid * 1280 + 640, 5, False)

  @pl.when(wid == NW - 1)
  def _():
    superchunk(jnp.int32((NW - 1) * 1280), 3, True)


def _k3_body(canvas_ref, ptr_ref, out_ref):
  for r in range(XPB):
    cv = canvas_ref[pl.ds(r * HP, HP), :C]
    pt = ptr_ref[pl.ds(r * HP, HP)]
    vals = jnp.where(pt[None, :] > 0, cv.T, jnp.float32(0.0))
    out_ref[0, :, r, :] = vals[:, :H]


@jax.jit
def kernel(PFN_output, pillar_tensor, batch_size):
  del batch_size
  key = ((pillar_tensor[:, 0] * W + pillar_tensor[:, 3]) * HP
         + pillar_tensor[:, 2]).astype(jnp.int32)
  featp = jnp.pad(PFN_output.T, ((0, CW - C), (0, 0))).T

  mesh = plsc.VectorSubcoreMesh(core_axis_name="c", subcore_axis_name="s")
  ptr = pl.kernel(
      _k1_body,
      out_type=jax.ShapeDtypeStruct((N,), jnp.int32),
      mesh=mesh,
      compiler_params=_SC_PARAMS,
      scratch_types=[
          pltpu.VMEM((KCH,), jnp.int32),
          pltpu.VMEM((RANGE,), jnp.int32),
      ],
  )(key)

  canvas = pl.kernel(
      _k2_body,
      out_type=jax.ShapeDtypeStruct((N + 8, CW), jnp.float32),
      mesh=mesh,
      compiler_params=_SC_PARAMS,
      scratch_types=[
          pltpu.VMEM((5, 128), jnp.int32),
          pltpu.VMEM((5, 128), jnp.int32),
          pltpu.VMEM((5, 128), jnp.int32),
          pltpu.VMEM((640, CW), jnp.float32),
          pltpu.SemaphoreType.DMA,
          pltpu.SemaphoreType.DMA,
          pltpu.SemaphoreType.DMA,
          pltpu.SemaphoreType.DMA,
      ],
  )(featp, key, ptr)

  out = pl.pallas_call(
      _k3_body,
      grid=(B, GPB),
      in_specs=[
          pl.BlockSpec((T, CW), lambda b, t: (b * GPB + t, 0)),
          pl.BlockSpec((T,), lambda b, t: (b * GPB + t,)),
      ],
      out_specs=pl.BlockSpec((1, C, XPB, H), lambda b, t: (b, 0, t, 0)),
      out_shape=jax.ShapeDtypeStruct((B, C, W, H), jnp.float32),
  )(canvas, ptr)

  return out.transpose(0, 1, 3, 2)


# speedup vs baseline: 1.0032x; 1.0032x over previous
import jax
import jax.numpy as jnp
from jax import lax
from jax.experimental import pallas as pl
from jax.experimental.pallas import tpu as pltpu
from jax.experimental.pallas import tpu_sc as plsc

C = 64
H = 496
W = 432
HP = 512
P = 40000
B = 4
N = B * W * HP
NC = 2
NS = 16
NW = NC * NS
RANGE = N // NW
KCH = 8000
NKCH = P // KCH
CW = 128
XPB = 24
T = XPB * HP
GPB = W // XPB
L = 16
DUMP = N

_SC_PARAMS = pltpu.CompilerParams(needs_layout_passes=False)


def _k1_body(key_hbm, ptr_hbm, keys_v, lmap):
  wid = lax.axis_index("s") * NC + lax.axis_index("c")
  kbase = wid * RANGE
  iota = lax.iota(jnp.int32, L)

  def zero_body(i, _):
    lmap[pl.ds(i * L, L)] = jnp.zeros((L,), jnp.int32)
    return 0
  lax.fori_loop(0, RANGE // L, zero_body, 0)

  GV = 10
  for ci in range(NKCH):
    pltpu.sync_copy(key_hbm.at[pl.ds(ci * KCH, KCH)], keys_v)

    def group_body(gi, _, ci=ci):
      def vreg(u, gi=gi):
        i = gi * GV + u
        k16 = keys_v[pl.ds(i * L, L)]
        pv = ci * KCH + i * L + iota + 1
        inr = (k16 >= kbase) & (k16 < kbase + RANGE)
        kk = jnp.where(inr, k16 - kbase, 0)
        return k16, pv, inr, kk

      acc = jnp.zeros((L,), jnp.bool_)
      for u in range(GV):
        _, pv, inr, kk = vreg(u)
        plsc.store_scatter(lmap, [kk], pv, mask=inr)
        g = plsc.load_gather(lmap, [kk])
        acc = acc | (inr & (g < pv))

      def fix(_):
        for u in range(GV):
          _, pv, inr, kk = vreg(u)

          def cond(lost_c):
            return jnp.any(lost_c)

          def step(lost_c, pv=pv, inr=inr, kk=kk):
            plsc.store_scatter(lmap, [kk], pv, mask=lost_c)
            g2 = plsc.load_gather(lmap, [kk])
            return inr & (g2 < pv)
          g = plsc.load_gather(lmap, [kk])
          lax.while_loop(cond, step, inr & (g < pv))
        return 0
      lax.cond(plsc.all_reduce_population_count(acc)[0] > 0,
               fix, lambda _: 0, 0)
      return 0
    lax.fori_loop(0, KCH // L // GV, group_body, 0)

  pltpu.sync_copy(lmap, ptr_hbm.at[pl.ds(kbase, RANGE)])


def _k2_body(feat_hbm, key_hbm, ptr_hbm, canvas_hbm,
             keys2, keys2b, w2, w2b, ridx2, ridx2b, featbuf,
             sem_k, sem_k2, sem_f, sem_g, sem_g2, sem_s):
  wid = lax.axis_index("s") * NC + lax.axis_index("c")
  iota = lax.iota(jnp.int32, L)

  def fire_keys(base, kref, sem):
    return [pltpu.async_copy(key_hbm.at[pl.ds(base + j * 128, 128)],
                             kref.at[j], sem) for j in range(5)]

  def fire_gathers(kref, wref, sem):
    return [pltpu.async_copy(ptr_hbm.at[kref.at[j]], wref.at[j], sem)
            for j in range(5)]

  def compute_ridx(base, kref, wref, rref):
    for j in range(5):
      for v in range(128 // L):
        k16 = kref[j, pl.ds(v * L, L)]
        w16 = wref[j, pl.ds(v * L, L)]
        p16 = base + j * 128 + v * L + iota + 1
        rref[j, pl.ds(v * L, L)] = jnp.where(w16 == p16, k16, DUMP)

  def fire_scatters(rref, sem):
    return [pltpu.async_copy(featbuf.at[pl.ds(j * 128, 128), :],
                             canvas_hbm.at[rref.at[j]], sem)
            for j in range(5)]

  @pl.when(wid < NW - 1)
  def _():
    b0 = wid * 1280
    b1 = b0 + 640
    kc0 = fire_keys(b0, keys2, sem_k)
    kc1 = fire_keys(b1, keys2b, sem_k2)
    cf0 = pltpu.async_copy(feat_hbm.at[pl.ds(b0, 640), :], featbuf, sem_f)
    for c in kc0:
      c.wait()
    cg0 = fire_gathers(keys2, w2, sem_g)
    for c in kc1:
      c.wait()
    cg1 = fire_gathers(keys2b, w2b, sem_g2)
    for c in cg0:
      c.wait()
    compute_ridx(b0, keys2, w2, ridx2)
    cf0.wait()
    cs0 = fire_scatters(ridx2, sem_s)
    for c in cg1:
      c.wait()
    compute_ridx(b1, keys2b, w2b, ridx2b)
    for c in cs0:
      c.wait()
    cf1 = pltpu.async_copy(feat_hbm.at[pl.ds(b1, 640), :], featbuf, sem_f)
    cf1.wait()
    cs1 = fire_scatters(ridx2b, sem_s)
    for c in cs1:
      c.wait()

  @pl.when(wid == NW - 1)
  def _():
    base = jnp.int32((NW - 1) * 1280)
    kc = [pltpu.async_copy(key_hbm.at[pl.ds(base + j * 128,
                                            64 if j == 2 else 128)],
                           keys2.at[j, pl.ds(0, 64 if j == 2 else 128)],
                           sem_k) for j in range(3)]
    for v in range(4):
      keys2[2, pl.ds(64 + v * L, L)] = jnp.zeros((L,), jnp.int32)
    cf = pltpu.async_copy(feat_hbm.at[pl.ds(base, 320), :],
                          featbuf.at[pl.ds(0, 320), :], sem_f)
    for c in kc:
      c.wait()
    cg = [pltpu.async_copy(ptr_hbm.at[keys2.at[j]], w2.at[j], sem_g)
          for j in range(3)]
    for c in cg:
      c.wait()
    for j in range(3):
      for v in range(128 // L):
        k16 = keys2[j, pl.ds(v * L, L)]
        w16 = w2[j, pl.ds(v * L, L)]
        p16 = base + j * 128 + v * L + iota + 1
        ridx2[j, pl.ds(v * L, L)] = jnp.where(w16 == p16, k16, DUMP)
    cf.wait()
    cs = [pltpu.async_copy(featbuf.at[pl.ds(j * 128, 128), :],
                           canvas_hbm.at[ridx2.at[j]], sem_s)
          for j in range(3)]
    for c in cs:
      c.wait()


def _k3_body(canvas_ref, ptr_ref, out_ref):
  for r in range(XPB):
    cv = canvas_ref[pl.ds(r * HP, HP), :C]
    pt = ptr_ref[pl.ds(r * HP, HP)]
    vals = jnp.where(pt[None, :] > 0, cv.T, jnp.float32(0.0))
    out_ref[0, :, r, :] = vals[:, :H]


@jax.jit
def kernel(PFN_output, pillar_tensor, batch_size):
  del batch_size
  key = ((pillar_tensor[:, 0] * W + pillar_tensor[:, 3]) * HP
         + pillar_tensor[:, 2]).astype(jnp.int32)
  featp = jnp.pad(PFN_output.T, ((0, CW - C), (0, 0))).T

  mesh = plsc.VectorSubcoreMesh(core_axis_name="c", subcore_axis_name="s")
  ptr = pl.kernel(
      _k1_body,
      out_type=jax.ShapeDtypeStruct((N,), jnp.int32),
      mesh=mesh,
      compiler_params=_SC_PARAMS,
      scratch_types=[
          pltpu.VMEM((KCH,), jnp.int32),
          pltpu.VMEM((RANGE,), jnp.int32),
      ],
  )(key)

  canvas = pl.kernel(
      _k2_body,
      out_type=jax.ShapeDtypeStruct((N + 8, CW), jnp.float32),
      mesh=mesh,
      compiler_params=_SC_PARAMS,
      scratch_types=[
          pltpu.VMEM((5, 128), jnp.int32),
          pltpu.VMEM((5, 128), jnp.int32),
          pltpu.VMEM((5, 128), jnp.int32),
          pltpu.VMEM((5, 128), jnp.int32),
          pltpu.VMEM((5, 128), jnp.int32),
          pltpu.VMEM((5, 128), jnp.int32),
          pltpu.VMEM((640, CW), jnp.float32),
          pltpu.SemaphoreType.DMA,
          pltpu.SemaphoreType.DMA,
          pltpu.SemaphoreType.DMA,
          pltpu.SemaphoreType.DMA,
          pltpu.SemaphoreType.DMA,
          pltpu.SemaphoreType.DMA,
      ],
  )(featp, key, ptr)

  out = pl.pallas_call(
      _k3_body,
      grid=(B, GPB),
      in_specs=[
          pl.BlockSpec((T, CW), lambda b, t: (b * GPB + t, 0)),
          pl.BlockSpec((T,), lambda b, t: (b * GPB + t,)),
      ],
      out_specs=pl.BlockSpec((1, C, XPB, H), lambda b, t: (b, 0, t, 0)),
      out_shape=jax.ShapeDtypeStruct((B, C, W, H), jnp.float32),
  )(canvas, ptr)

  return out.transpose(0, 1, 3, 2)
